# baseline (speedup 1.0000x reference)
import functools

import jax
import jax.numpy as jnp
from jax import lax
from jax.experimental import pallas as pl
from jax.experimental.pallas import tpu as pltpu
from jax.experimental.pallas import tpu_sc as plsc

NC = 2
NS = 16
NW = NC * NS
C = 128


def _seg_sum_sc(h, src_idx2d, dst_idx2d, zeros_blk, n_acc, d, cp0, cp1,
                tc_tiling=True):
  rows_per_tile = n_acc // NS
  zchunks = rows_per_tile // C
  mesh = plsc.VectorSubcoreMesh(core_axis_name="c", subcore_axis_name="s")

  cph = 32

  @functools.partial(
      pl.kernel,
      mesh=mesh,
      out_type=jax.ShapeDtypeStruct((NC * n_acc, d), jnp.float32),
      scratch_types=[
          pltpu.VMEM((cph, C), jnp.int32),
          pltpu.VMEM((cph, C), jnp.int32),
          pltpu.VMEM((C, d), jnp.float32),
          pltpu.VMEM((C, d), jnp.float32),
          pltpu.SemaphoreType.DMA,
          pltpu.SemaphoreType.DMA,
          pltpu.VMEM_SHARED((n_acc, d), jnp.float32),
      ],
      compiler_params=(None if tc_tiling else
                       pltpu.CompilerParams(use_tc_tiling_on_sc=False)),
  )
  def seg_kernel(h_hbm, src_hbm, dst_hbm, z_hbm, out_hbm, src_v, dst_v, rows0,
                 rows1, sem0, sem1, acc_sh):
    cid = lax.axis_index("c")
    sid = lax.axis_index("s")
    for z in range(zchunks):
      pltpu.sync_copy(z_hbm, acc_sh.at[pl.ds(sid * rows_per_tile + z * C, C)])
    plsc.subcore_barrier()

    def run_edges(tile_base, cpt):
      for ph in range(cpt // cph):
        base = tile_base + ph * cph
        pltpu.sync_copy(src_hbm.at[pl.ds(base, cph)], src_v)
        pltpu.sync_copy(dst_hbm.at[pl.ds(base, cph)], dst_v)
        pltpu.make_async_copy(h_hbm.at[src_v.at[0]], rows0, sem0).start()

        def body(j2, carry):
          for b, (rb_, sb_, ro_, so_) in ((0, (rows0, sem0, rows1, sem1)),
                                          (1, (rows1, sem1, rows0, sem0))):
            j = j2 * 2 + b

            @pl.when(j + 1 < cph)
            def _():
              pltpu.make_async_copy(h_hbm.at[src_v.at[j + 1]], ro_,
                                    so_).start()

            pltpu.make_async_copy(h_hbm.at[src_v.at[j]], rb_, sb_).wait()
            pltpu.sync_copy(rb_, acc_sh.at[dst_v.at[j]], add=True)
          return carry

        lax.fori_loop(0, cph // 2, body, 0)

    @pl.when(cid == 0)
    def _():
      run_edges(sid * cp0, cp0)

    @pl.when(cid == 1)
    def _():
      run_edges(NS * cp0 + sid * cp1, cp1)

    plsc.subcore_barrier()
    for z in range(zchunks):
      r = sid * rows_per_tile + z * C
      pltpu.sync_copy(acc_sh.at[pl.ds(r, C)],
                      out_hbm.at[pl.ds(cid * n_acc + r, C)])

  return seg_kernel(h, src_idx2d, dst_idx2d, zeros_blk)


def _deg_sc(idx2d, zeros_blk, iota_rows, n_rows, chunks_per_tile):
  hr = n_rows // C
  half = iota_rows.shape[1]
  niota = hr // half
  nread = hr // 16
  mesh = plsc.VectorSubcoreMesh(core_axis_name="c", subcore_axis_name="s")

  @functools.partial(
      pl.kernel,
      mesh=mesh,
      out_type=jax.ShapeDtypeStruct((NC * hr, C), jnp.float32),
      scratch_types=[
          pltpu.VMEM((chunks_per_tile, C), jnp.int32),
          pltpu.VMEM((hr, C), jnp.float32),
          pltpu.VMEM((8, half), jnp.int32),
          pltpu.VMEM_SHARED((hr, C), jnp.float32),
      ],
      compiler_params=pltpu.CompilerParams(needs_layout_passes=False),
  )
  def deg_kernel(idx_hbm, z_hbm, iota_hbm, out_hbm, idx_v, hist_v, iota_v,
                 acc_sh):
    cid = lax.axis_index("c")
    sid = lax.axis_index("s")
    wid = cid * NS + sid
    pltpu.sync_copy(z_hbm, hist_v)
    @pl.when(sid < nread)
    def _():
      pltpu.sync_copy(hist_v.at[pl.ds(0, 16)],
                      acc_sh.at[pl.ds(sid * 16, 16)])
    pltpu.sync_copy(iota_hbm, iota_v)
    pltpu.sync_copy(idx_hbm.at[pl.ds(wid * chunks_per_tile, chunks_per_tile)],
                    idx_v)
    plsc.subcore_barrier()

    ones16 = jnp.full((16,), 1.0, jnp.float32)

    def body(j, carry):
      for cc in range(C // 16):
        iv = idx_v[j, pl.ds(cc * 16, 16)]
        r = lax.shift_right_logical(iv, 7)
        col = lax.bitwise_and(iv, 127)
        plsc.addupdate_scatter(hist_v, [r, col], ones16)
      return carry

    lax.fori_loop(0, chunks_per_tile, body, 0)
    for t in range(niota):
      pltpu.sync_copy(hist_v.at[pl.ds(t * half, half)],
                      acc_sh.at[iota_v.at[t]], add=True)
    plsc.subcore_barrier()

    @pl.when(sid < nread)
    def _():
      r = sid * 16
      pltpu.sync_copy(acc_sh.at[pl.ds(r, 16)],
                      out_hbm.at[pl.ds(cid * hr + r, 16)])

  return deg_kernel(idx2d, zeros_blk, iota_rows)


def _tc_scale_matmul(feat, w, dsrc_t, rb):
  n, fd = feat.shape
  h = w.shape[1]

  def body(f_ref, w_ref, d_ref, o_ref):
    d = jnp.sum(d_ref[...], axis=1, keepdims=True)
    nsrc = lax.rsqrt(jnp.maximum(d, 1.0))
    o_ref[...] = jnp.dot(f_ref[...], w_ref[...],
                         preferred_element_type=jnp.float32) * nsrc

  return pl.pallas_call(
      body,
      grid=(n // rb,),
      in_specs=[
          pl.BlockSpec((rb, fd), lambda i: (i, 0)),
          pl.BlockSpec((fd, h), lambda i: (0, 0)),
          pl.BlockSpec((rb, NC), lambda i: (i, 0)),
      ],
      out_specs=pl.BlockSpec((rb, h), lambda i: (i, 0)),
      out_shape=jax.ShapeDtypeStruct((n, h), jnp.float32),
  )(feat, w, dsrc_t)


def _tc_mid(agg1, ddst_t, dsrc_t, b1, w2, rb):
  _, n, h = agg1.shape
  k = w2.shape[1]

  def body(a_ref, dd_ref, ds_ref, b_ref, w_ref, o_ref):
    agg = a_ref[0] + a_ref[1]
    dd = jnp.sum(dd_ref[...], axis=1, keepdims=True)
    ndst = lax.rsqrt(jnp.maximum(dd, 1.0))
    h1 = jnp.maximum(agg * ndst + b_ref[...], 0.0)
    ds = jnp.sum(ds_ref[...], axis=1, keepdims=True)
    nsrc = lax.rsqrt(jnp.maximum(ds, 1.0))
    o_ref[...] = jnp.dot(h1, w_ref[...],
                         preferred_element_type=jnp.float32) * nsrc

  return pl.pallas_call(
      body,
      grid=(n // rb,),
      in_specs=[
          pl.BlockSpec((NC, rb, h), lambda i: (0, i, 0)),
          pl.BlockSpec((rb, NC), lambda i: (i, 0)),
          pl.BlockSpec((rb, NC), lambda i: (i, 0)),
          pl.BlockSpec((1, h), lambda i: (0, 0)),
          pl.BlockSpec((h, k), lambda i: (0, 0)),
      ],
      out_specs=pl.BlockSpec((rb, k), lambda i: (i, 0)),
      out_shape=jax.ShapeDtypeStruct((n, k), jnp.float32),
  )(agg1, ddst_t, dsrc_t, b1, w2)


def _tc_final(agg2, ddst_t, b2, rb):
  _, n, k = agg2.shape
  kout = b2.shape[1]

  def body(a_ref, dd_ref, b_ref, o_ref):
    agg = a_ref[0, :, :kout] + a_ref[1, :, :kout]
    dd = jnp.sum(dd_ref[...], axis=1, keepdims=True)
    ndst = lax.rsqrt(jnp.maximum(dd, 1.0))
    o_ref[...] = agg * ndst + b_ref[...]

  return pl.pallas_call(
      body,
      grid=(n // rb,),
      in_specs=[
          pl.BlockSpec((NC, rb, k), lambda i: (0, i, 0)),
          pl.BlockSpec((rb, NC), lambda i: (i, 0)),
          pl.BlockSpec((1, kout), lambda i: (0, 0)),
      ],
      out_specs=pl.BlockSpec((rb, kout), lambda i: (i, 0)),
      out_shape=jax.ShapeDtypeStruct((n, kout), jnp.float32),
  )(agg2, ddst_t, b2)


def kernel(feat, edge_index, W1, b1, W2, b2):
  n, fd = feat.shape
  e = edge_index.shape[1]
  h = W1.shape[1]
  k = W2.shape[1]

  n_acc = -(-(n + 1) // (NS * C)) * (NS * C)
  dummy = n
  chunks_per_tile = -(-(-(-e // (NW * C))) // 8) * 8
  e_pad = NW * chunks_per_tile * C
  pad = e_pad - e
  cp0, cp1 = 96, 64

  src = edge_index[0].astype(jnp.int32)
  dst = edge_index[1].astype(jnp.int32)
  pad_dummy = jnp.full((pad,), dummy, jnp.int32)
  src_g = jnp.concatenate([src, jnp.zeros((pad,), jnp.int32)]).reshape(-1, C)
  dst_g = jnp.concatenate([dst, pad_dummy]).reshape(-1, C)
  deg_idx = jnp.concatenate([
      jnp.concatenate([src, pad_dummy]),
      jnp.concatenate([dst, pad_dummy]) + n_acc,
  ]).reshape(-1, C)
  deg_chunks_per_tile = 2 * chunks_per_tile

  zeros_h = jnp.zeros((C, h), jnp.float32)
  zeros_kp = jnp.zeros((C, k), jnp.float32)

  n_rows = 2 * n_acc
  hr = n_rows // C
  half = hr // 2
  iota_rows = jnp.concatenate([
      jnp.arange(hr, dtype=jnp.int32).reshape(2, half),
      jnp.zeros((6, half), jnp.int32),
  ])
  zeros_hr = jnp.zeros((hr, C), jnp.float32)
  dpar = _deg_sc(deg_idx, zeros_hr, iota_rows, n_rows, deg_chunks_per_tile)
  dflat = dpar.reshape(NC, n_rows)
  dsrc_t = dflat[:, :n].T
  ddst_t = dflat[:, n_acc:n_acc + n].T

  rb = 1000
  h1s = _tc_scale_matmul(feat, W1, dsrc_t, rb)
  agg1 = _seg_sum_sc(h1s, src_g, dst_g, zeros_h, n_acc, h, cp0, cp1,
                     tc_tiling=False)
  agg1 = agg1.reshape(NC, n_acc, h)[:, :n, :]
  h2s = _tc_mid(agg1, ddst_t, dsrc_t, b1.reshape(1, h), W2, rb)
  agg2 = _seg_sum_sc(h2s, src_g, dst_g, zeros_kp, n_acc, k, cp0, cp1,
                     tc_tiling=False)
  agg2 = agg2.reshape(NC, n_acc, k)[:, :n, :]
  return _tc_final(agg2, ddst_t, b2.reshape(1, k), rb)


# speedup vs baseline: 1.1513x; 1.1513x over previous
import functools

import jax
import jax.numpy as jnp
from jax import lax
from jax.experimental import pallas as pl
from jax.experimental.pallas import tpu as pltpu
from jax.experimental.pallas import tpu_sc as plsc

NC = 2
NS = 16
NW = NC * NS
C = 128


def _seg_sum_sc(h, src_idx2d, dst_idx2d, zeros_blk, n_acc, d, cp0, cp1,
                tc_tiling=True):
  rows_per_tile = n_acc // NS
  zchunks = rows_per_tile // C
  mesh = plsc.VectorSubcoreMesh(core_axis_name="c", subcore_axis_name="s")

  cph = 16

  @functools.partial(
      pl.kernel,
      mesh=mesh,
      out_type=jax.ShapeDtypeStruct((NC * n_acc, d), jnp.float32),
      scratch_types=[
          pltpu.VMEM((cph, C), jnp.int32),
          pltpu.VMEM((cph, C), jnp.int32),
          pltpu.VMEM((C, d), jnp.float32),
          pltpu.VMEM((C, d), jnp.float32),
          pltpu.SemaphoreType.DMA,
          pltpu.SemaphoreType.DMA,
          pltpu.VMEM_SHARED((n_acc, d), jnp.float32),
      ],
      compiler_params=(None if tc_tiling else
                       pltpu.CompilerParams(use_tc_tiling_on_sc=False)),
  )
  def seg_kernel(h_hbm, src_hbm, dst_hbm, z_hbm, out_hbm, src_v, dst_v, rows0,
                 rows1, sem0, sem1, acc_sh):
    cid = lax.axis_index("c")
    sid = lax.axis_index("s")
    for z in range(zchunks):
      pltpu.sync_copy(z_hbm, acc_sh.at[pl.ds(sid * rows_per_tile + z * C, C)])
    plsc.subcore_barrier()

    def run_edges(tile_base, cpt):
      for ph in range(cpt // cph):
        base = tile_base + ph * cph
        pltpu.sync_copy(src_hbm.at[pl.ds(base, cph)], src_v)
        pltpu.sync_copy(dst_hbm.at[pl.ds(base, cph)], dst_v)
        pltpu.make_async_copy(h_hbm.at[src_v.at[0]], rows0, sem0).start()

        def body(j2, carry):
          for b, (rb_, sb_, ro_, so_) in ((0, (rows0, sem0, rows1, sem1)),
                                          (1, (rows1, sem1, rows0, sem0))):
            j = j2 * 2 + b

            @pl.when(j + 1 < cph)
            def _():
              pltpu.make_async_copy(h_hbm.at[src_v.at[j + 1]], ro_,
                                    so_).start()

            pltpu.make_async_copy(h_hbm.at[src_v.at[j]], rb_, sb_).wait()
            pltpu.sync_copy(rb_, acc_sh.at[dst_v.at[j]], add=True)
          return carry

        lax.fori_loop(0, cph // 2, body, 0)

    @pl.when(cid == 0)
    def _():
      run_edges(sid * cp0, cp0)

    @pl.when(cid == 1)
    def _():
      run_edges(NS * cp0 + sid * cp1, cp1)

    plsc.subcore_barrier()
    for z in range(zchunks):
      r = sid * rows_per_tile + z * C
      pltpu.sync_copy(acc_sh.at[pl.ds(r, C)],
                      out_hbm.at[pl.ds(cid * n_acc + r, C)])

  return seg_kernel(h, src_idx2d, dst_idx2d, zeros_blk)


def _deg_sc(idx2d, zeros_blk, iota_rows, n_rows, chunks_per_tile):
  hr = n_rows // C
  half = iota_rows.shape[1]
  niota = hr // half
  nread = hr // 16
  mesh = plsc.VectorSubcoreMesh(core_axis_name="c", subcore_axis_name="s")

  @functools.partial(
      pl.kernel,
      mesh=mesh,
      out_type=jax.ShapeDtypeStruct((NC * hr, C), jnp.float32),
      scratch_types=[
          pltpu.VMEM((chunks_per_tile, C), jnp.int32),
          pltpu.VMEM((hr, C), jnp.float32),
          pltpu.VMEM((8, half), jnp.int32),
          pltpu.VMEM_SHARED((hr, C), jnp.float32),
      ],
      compiler_params=pltpu.CompilerParams(needs_layout_passes=False),
  )
  def deg_kernel(idx_hbm, z_hbm, iota_hbm, out_hbm, idx_v, hist_v, iota_v,
                 acc_sh):
    cid = lax.axis_index("c")
    sid = lax.axis_index("s")
    wid = cid * NS + sid
    pltpu.sync_copy(z_hbm, hist_v)
    @pl.when(sid < nread)
    def _():
      pltpu.sync_copy(hist_v.at[pl.ds(0, 16)],
                      acc_sh.at[pl.ds(sid * 16, 16)])
    pltpu.sync_copy(iota_hbm, iota_v)
    pltpu.sync_copy(idx_hbm.at[pl.ds(wid * chunks_per_tile, chunks_per_tile)],
                    idx_v)
    plsc.subcore_barrier()

    ones16 = jnp.full((16,), 1.0, jnp.float32)

    def body(j, carry):
      for cc in range(C // 16):
        iv = idx_v[j, pl.ds(cc * 16, 16)]
        r = lax.shift_right_logical(iv, 7)
        col = lax.bitwise_and(iv, 127)
        plsc.addupdate_scatter(hist_v, [r, col], ones16)
      return carry

    lax.fori_loop(0, chunks_per_tile, body, 0)
    for t in range(niota):
      pltpu.sync_copy(hist_v.at[pl.ds(t * half, half)],
                      acc_sh.at[iota_v.at[t]], add=True)
    plsc.subcore_barrier()

    @pl.when(sid < nread)
    def _():
      r = sid * 16
      pltpu.sync_copy(acc_sh.at[pl.ds(r, 16)],
                      out_hbm.at[pl.ds(cid * hr + r, 16)])

  return deg_kernel(idx2d, zeros_blk, iota_rows)


def _tc_scale_matmul(feat, w, dsrc_t, rb):
  n, fd = feat.shape
  h = w.shape[1]

  def body(f_ref, w_ref, d_ref, o_ref):
    d = jnp.sum(d_ref[...], axis=1, keepdims=True)
    nsrc = lax.rsqrt(jnp.maximum(d, 1.0))
    o_ref[...] = jnp.dot(f_ref[...], w_ref[...],
                         preferred_element_type=jnp.float32) * nsrc

  return pl.pallas_call(
      body,
      grid=(n // rb,),
      in_specs=[
          pl.BlockSpec((rb, fd), lambda i: (i, 0)),
          pl.BlockSpec((fd, h), lambda i: (0, 0)),
          pl.BlockSpec((rb, NC), lambda i: (i, 0)),
      ],
      out_specs=pl.BlockSpec((rb, h), lambda i: (i, 0)),
      out_shape=jax.ShapeDtypeStruct((n, h), jnp.float32),
  )(feat, w, dsrc_t)


def _tc_mid(agg1, ddst_t, dsrc_t, b1, w2, rb):
  _, n, h = agg1.shape
  k = w2.shape[1]

  def body(a_ref, dd_ref, ds_ref, b_ref, w_ref, o_ref):
    agg = a_ref[0] + a_ref[1]
    dd = jnp.sum(dd_ref[...], axis=1, keepdims=True)
    ndst = lax.rsqrt(jnp.maximum(dd, 1.0))
    h1 = jnp.maximum(agg * ndst + b_ref[...], 0.0)
    ds = jnp.sum(ds_ref[...], axis=1, keepdims=True)
    nsrc = lax.rsqrt(jnp.maximum(ds, 1.0))
    o_ref[...] = jnp.dot(h1, w_ref[...],
                         preferred_element_type=jnp.float32) * nsrc

  return pl.pallas_call(
      body,
      grid=(n // rb,),
      in_specs=[
          pl.BlockSpec((NC, rb, h), lambda i: (0, i, 0)),
          pl.BlockSpec((rb, NC), lambda i: (i, 0)),
          pl.BlockSpec((rb, NC), lambda i: (i, 0)),
          pl.BlockSpec((1, h), lambda i: (0, 0)),
          pl.BlockSpec((h, k), lambda i: (0, 0)),
      ],
      out_specs=pl.BlockSpec((rb, k), lambda i: (i, 0)),
      out_shape=jax.ShapeDtypeStruct((n, k), jnp.float32),
  )(agg1, ddst_t, dsrc_t, b1, w2)


def _tc_final(agg2, ddst_t, b2, rb):
  _, n, k = agg2.shape
  kout = b2.shape[1]

  def body(a_ref, dd_ref, b_ref, o_ref):
    agg = a_ref[0, :, :kout] + a_ref[1, :, :kout]
    dd = jnp.sum(dd_ref[...], axis=1, keepdims=True)
    ndst = lax.rsqrt(jnp.maximum(dd, 1.0))
    o_ref[...] = agg * ndst + b_ref[...]

  return pl.pallas_call(
      body,
      grid=(n // rb,),
      in_specs=[
          pl.BlockSpec((NC, rb, k), lambda i: (0, i, 0)),
          pl.BlockSpec((rb, NC), lambda i: (i, 0)),
          pl.BlockSpec((1, kout), lambda i: (0, 0)),
      ],
      out_specs=pl.BlockSpec((rb, kout), lambda i: (i, 0)),
      out_shape=jax.ShapeDtypeStruct((n, kout), jnp.float32),
  )(agg2, ddst_t, b2)


def kernel(feat, edge_index, W1, b1, W2, b2):
  n, fd = feat.shape
  e = edge_index.shape[1]
  h = W1.shape[1]
  k = W2.shape[1]

  n_acc = -(-(n + 1) // (NS * C)) * (NS * C)
  dummy = n
  chunks_per_tile = -(-(-(-e // (NW * C))) // 8) * 8
  e_pad = NW * chunks_per_tile * C
  pad = e_pad - e
  cp0, cp1 = 144, 16

  src = edge_index[0].astype(jnp.int32)
  dst = edge_index[1].astype(jnp.int32)
  pad_dummy = jnp.full((pad,), dummy, jnp.int32)
  src_g = jnp.concatenate([src, jnp.zeros((pad,), jnp.int32)]).reshape(-1, C)
  dst_g = jnp.concatenate([dst, pad_dummy]).reshape(-1, C)
  deg_idx = jnp.concatenate([
      jnp.concatenate([src, pad_dummy]),
      jnp.concatenate([dst, pad_dummy]) + n_acc,
  ]).reshape(-1, C)
  deg_chunks_per_tile = 2 * chunks_per_tile

  zeros_h = jnp.zeros((C, h), jnp.float32)
  zeros_kp = jnp.zeros((C, k), jnp.float32)

  n_rows = 2 * n_acc
  hr = n_rows // C
  half = hr // 2
  iota_rows = jnp.concatenate([
      jnp.arange(hr, dtype=jnp.int32).reshape(2, half),
      jnp.zeros((6, half), jnp.int32),
  ])
  zeros_hr = jnp.zeros((hr, C), jnp.float32)
  dpar = _deg_sc(deg_idx, zeros_hr, iota_rows, n_rows, deg_chunks_per_tile)
  dflat = dpar.reshape(NC, n_rows)
  dsrc_t = dflat[:, :n].T
  ddst_t = dflat[:, n_acc:n_acc + n].T

  rb = 1000
  h1s = _tc_scale_matmul(feat, W1, dsrc_t, rb)
  agg1 = _seg_sum_sc(h1s, src_g, dst_g, zeros_h, n_acc, h, cp0, cp1,
                     tc_tiling=False)
  agg1 = agg1.reshape(NC, n_acc, h)[:, :n, :]
  h2s = _tc_mid(agg1, ddst_t, dsrc_t, b1.reshape(1, h), W2, rb)
  agg2 = _seg_sum_sc(h2s, src_g, dst_g, zeros_kp, n_acc, k, cp0, cp1,
                     tc_tiling=False)
  agg2 = agg2.reshape(NC, n_acc, k)[:, :n, :]
  return _tc_final(agg2, ddst_t, b2.reshape(1, k), rb)
